# baseline (speedup 1.0000x reference)
import functools

import jax
import jax.numpy as jnp
import numpy as np
from jax.experimental import pallas as pl
from jax.experimental.pallas import tpu as pltpu
from jax.experimental.pallas import tpu_sc as plsc

B, T, C = 1, 2048, 768
NR, NT = 6, 2
NE = NR + NT
HID = 2 * C

CAP = 640
G = NE * CAP
BT = 1024
NW = 32

_F32 = jnp.float32
_BF16 = jnp.bfloat16


def _routing():
    w = np.asarray(jax.random.randint(jax.random.key(42), (B, T, 2), 0, NE))[0]
    gidx = np.arange(G, dtype=np.int32) % (2 * T)
    inv = np.zeros((2 * T,), np.int32)
    fill = [0] * NE
    for s in range(2):
        for t in range(T):
            e = int(w[t, s])
            row = e * CAP + fill[e]
            fill[e] += 1
            gidx[row] = t + T * (e >= NR)
            inv[s * T + t] = row
    return gidx, inv


_GIDX, _INV = _routing()


def _ln(x, g, b):
    m = x.mean(-1, keepdims=True)
    v = ((x - m) ** 2).mean(-1, keepdims=True)
    return (x - m) / jnp.sqrt(v + 1e-5) * g + b


def _dot(a, b):
    return jnp.dot(a.astype(_BF16), b.astype(_BF16),
                   preferred_element_type=_F32)


def _prologue_body(x_ref, wr_ref, wk_ref, wv_ref, wo_ref, wb1_ref, wb2_ref,
                   g1_ref, b1_ref, g2_ref, b2_ref, mur_ref, muk_ref, muv_ref,
                   bb_ref, x1_ref, v_ref, hh_ref, carry_ref):
    i = pl.program_id(0)
    x = x_ref[...]
    a = _ln(x, g1_ref[...].reshape(1, C), b1_ref[...].reshape(1, C))
    prev = jnp.where(i == 0, 0.0, carry_ref[...])
    ap = pltpu.roll(a, 1, axis=0)
    row0 = jax.lax.broadcasted_iota(jnp.int32, (BT, 1), 0) == 0
    ap = jnp.where(row0, prev, ap)
    carry_ref[...] = a[BT - 1:BT, :]
    xx = ap - a
    r = _dot(a + xx * mur_ref[...].reshape(1, C), wr_ref[...])
    k = _dot(a + xx * muk_ref[...].reshape(1, C), wk_ref[...])
    v = _dot(a + xx * muv_ref[...].reshape(1, C), wv_ref[...])
    att = _dot(jax.nn.sigmoid(r) * v, wo_ref[...])
    x1 = x + att
    h = _ln(x1, g2_ref[...].reshape(1, C), b2_ref[...].reshape(1, C))
    prefix = jnp.tanh(_dot(h, wb1_ref[...]) + _dot(k, wb2_ref[...])
                      + bb_ref[...].reshape(1, C))
    x1_ref[...] = x1
    v_ref[...] = v
    hh_ref[0] = h
    hh_ref[1] = h + prefix


def _prologue(x2d, p):
    blk = pl.BlockSpec((BT, C), lambda i: (i, 0))
    wspec = pl.BlockSpec((C, C), lambda i: (0, 0))
    vspec = pl.BlockSpec((C,), lambda i: (0,))
    vecs = [p[n] for n in
            ('ln1_g', 'ln1_b', 'ln2_g', 'ln2_b', 'mu_r', 'mu_k', 'mu_v', 'bb')]
    return pl.pallas_call(
        _prologue_body,
        grid=(T // BT,),
        in_specs=[blk, wspec, wspec, wspec, wspec,
                  pl.BlockSpec((C, C), lambda i: (0, 0)),
                  pl.BlockSpec((C, C), lambda i: (1, 0))] + [vspec] * 8,
        out_specs=[blk, blk, pl.BlockSpec((2, BT, C), lambda i: (0, i, 0))],
        out_shape=[jax.ShapeDtypeStruct((T, C), _F32),
                   jax.ShapeDtypeStruct((T, C), _F32),
                   jax.ShapeDtypeStruct((2, T, C), _F32)],
        scratch_shapes=[pltpu.VMEM((1, C), _F32)],
    )(x2d, p['Wr'], p['Wk'], p['Wv'], p['Wo'], p['Wb'], p['Wb'], *vecs)


def _ffn_body(x_ref, w1_ref, b1_ref, w2_ref, b2_ref, y_ref):
    e = pl.program_id(0)
    w1 = w1_ref[0].astype(_BF16)
    w2 = w2_ref[0].astype(_BF16)
    hid = _dot(x_ref[...], w1) + b1_ref[pl.ds(e, 1), :]
    act = jnp.where(e < NR, jnp.square(jax.nn.relu(hid)), jax.nn.gelu(hid))
    y_ref[...] = _dot(act, w2) + b2_ref[pl.ds(e, 1), :]


def _ffn(x_rows, w1, b1, w2, b2):
    return pl.pallas_call(
        _ffn_body,
        grid=(NE,),
        in_specs=[
            pl.BlockSpec((CAP, C), lambda e: (e, 0)),
            pl.BlockSpec((1, C, HID), lambda e: (e, 0, 0)),
            pl.BlockSpec((NE, HID), lambda e: (0, 0)),
            pl.BlockSpec((1, HID, C), lambda e: (e, 0, 0)),
            pl.BlockSpec((NE, C), lambda e: (0, 0)),
        ],
        out_specs=pl.BlockSpec((CAP, C), lambda e: (e, 0)),
        out_shape=jax.ShapeDtypeStruct((G, C), _F32),
    )(x_rows, w1, b1, w2, b2)


def _combine_body(x1_ref, d_ref, o_ref):
    o_ref[...] = x1_ref[...] + 0.5 * (d_ref[0] + d_ref[1])


def _combine(x1, d):
    blk = pl.BlockSpec((BT, C), lambda i: (i, 0))
    return pl.pallas_call(
        _combine_body,
        grid=(T // BT,),
        in_specs=[blk, pl.BlockSpec((2, BT, C), lambda i: (0, i, 0))],
        out_specs=blk,
        out_shape=jax.ShapeDtypeStruct((T, C), _F32),
    )(x1, d.reshape(2, T, C))


def _sc_gather(data, idx, n_rows):
    width = data.shape[1]
    bpw = n_rows // NW
    mesh = plsc.VectorSubcoreMesh(core_axis_name="c", subcore_axis_name="s")

    @functools.partial(
        pl.kernel, mesh=mesh,
        out_type=jax.ShapeDtypeStruct((n_rows, width), data.dtype),
        scratch_types=[pltpu.VMEM((bpw,), jnp.int32),
                       pltpu.VMEM((bpw, width), data.dtype),
                       pltpu.SemaphoreType.DMA])
    def k(data_hbm, idx_hbm, out_hbm, idx_v, rows_v, sem):
        wid = jax.lax.axis_index("s") * 2 + jax.lax.axis_index("c")
        base = wid * bpw
        pltpu.sync_copy(idx_hbm.at[pl.ds(base, bpw)], idx_v)
        pltpu.async_copy(data_hbm.at[idx_v], rows_v, sem).wait()
        pltpu.sync_copy(rows_v, out_hbm.at[pl.ds(base, bpw)])

    return k(data, idx)


def kernel(x, v_first, capital_shares, params, step, warmup_steps, use_market,
           training):
    p = params
    x2d = x.reshape(T, C)

    x1, v, hh = _prologue(x2d, p)

    xg = _sc_gather(hh.reshape(2 * T, C), jnp.asarray(_GIDX), G)

    y = _ffn(xg, p['W1'], p['b1'], p['W2'], p['b2'])

    d = _sc_gather(y, jnp.asarray(_INV), 2 * T)

    x2 = _combine(x1, d)
    return (x2.reshape(B, T, C), v.reshape(B, T, C))


# speedup vs baseline: 1.0115x; 1.0115x over previous
import functools

import jax
import jax.numpy as jnp
import numpy as np
from jax.experimental import pallas as pl
from jax.experimental.pallas import tpu as pltpu
from jax.experimental.pallas import tpu_sc as plsc

B, T, C = 1, 2048, 768
NR, NT = 6, 2
NE = NR + NT
HID = 2 * C

CAP = 576
G = NE * CAP
BT = 512
NW = 32

_F32 = jnp.float32
_BF16 = jnp.bfloat16


def _routing():
    w = np.asarray(jax.random.randint(jax.random.key(42), (B, T, 2), 0, NE))[0]
    gidx = np.arange(G, dtype=np.int32) % (2 * T)
    inv = np.zeros((2 * T,), np.int32)
    fill = [0] * NE
    for s in range(2):
        for t in range(T):
            e = int(w[t, s])
            row = e * CAP + fill[e]
            fill[e] += 1
            gidx[row] = t + T * (e >= NR)
            inv[s * T + t] = row
    return gidx, inv


_GIDX, _INV = _routing()
NEW_ = NE // 2
HALF = NEW_ * CAP
_GIDX_A, _GIDX_B = _GIDX[:HALF].copy(), _GIDX[HALF:].copy()


def _ln(x, g, b):
    m = x.mean(-1, keepdims=True)
    v = ((x - m) ** 2).mean(-1, keepdims=True)
    return (x - m) / jnp.sqrt(v + 1e-5) * g + b


def _dot(a, b):
    return jnp.dot(a.astype(_BF16), b.astype(_BF16),
                   preferred_element_type=_F32)


def _prologue_body(x_ref, wr_ref, wk_ref, wv_ref, wo_ref, wb1_ref, wb2_ref,
                   g1_ref, b1_ref, g2_ref, b2_ref, mur_ref, muk_ref, muv_ref,
                   bb_ref, x1_ref, v_ref, hh_ref, carry_ref):
    i = pl.program_id(0)
    x = x_ref[...]
    a = _ln(x, g1_ref[...].reshape(1, C), b1_ref[...].reshape(1, C))
    prev = jnp.where(i == 0, 0.0, carry_ref[...])
    ap = pltpu.roll(a, 1, axis=0)
    row0 = jax.lax.broadcasted_iota(jnp.int32, (BT, 1), 0) == 0
    ap = jnp.where(row0, prev, ap)
    carry_ref[...] = a[BT - 1:BT, :]
    xx = ap - a
    r = _dot(a + xx * mur_ref[...].reshape(1, C), wr_ref[...])
    k = _dot(a + xx * muk_ref[...].reshape(1, C), wk_ref[...])
    v = _dot(a + xx * muv_ref[...].reshape(1, C), wv_ref[...])
    att = _dot(jax.nn.sigmoid(r) * v, wo_ref[...])
    x1 = x + att
    h = _ln(x1, g2_ref[...].reshape(1, C), b2_ref[...].reshape(1, C))
    prefix = jnp.tanh(_dot(h, wb1_ref[...]) + _dot(k, wb2_ref[...])
                      + bb_ref[...].reshape(1, C))
    x1_ref[...] = x1
    v_ref[...] = v
    hh_ref[0] = h
    hh_ref[1] = h + prefix


def _prologue(x2d, p):
    blk = pl.BlockSpec((BT, C), lambda i: (i, 0))
    wspec = pl.BlockSpec((C, C), lambda i: (0, 0))
    vspec = pl.BlockSpec((C,), lambda i: (0,))
    vecs = [p[n] for n in
            ('ln1_g', 'ln1_b', 'ln2_g', 'ln2_b', 'mu_r', 'mu_k', 'mu_v', 'bb')]
    return pl.pallas_call(
        _prologue_body,
        grid=(T // BT,),
        in_specs=[blk, wspec, wspec, wspec, wspec,
                  pl.BlockSpec((C, C), lambda i: (0, 0)),
                  pl.BlockSpec((C, C), lambda i: (1, 0))] + [vspec] * 8,
        out_specs=[blk, blk, pl.BlockSpec((2, BT, C), lambda i: (0, i, 0))],
        out_shape=[jax.ShapeDtypeStruct((T, C), _F32),
                   jax.ShapeDtypeStruct((T, C), _F32),
                   jax.ShapeDtypeStruct((2, T, C), _F32)],
        scratch_shapes=[pltpu.VMEM((1, C), _F32)],
    )(x2d, p['Wr'], p['Wk'], p['Wv'], p['Wo'], p['Wb'], p['Wb'], *vecs)


def _ffn_wave_body(off, x_ref, w1_ref, b1_ref, w2_ref, b2_ref, y_ref):
    e = pl.program_id(0) + off
    w1 = w1_ref[0].astype(_BF16)
    w2 = w2_ref[0].astype(_BF16)
    hid = _dot(x_ref[...], w1) + b1_ref[pl.ds(e, 1), :]
    if off + NEW_ <= NR:
        act = jnp.square(jax.nn.relu(hid))
    else:
        act = jnp.where(e < NR, jnp.square(jax.nn.relu(hid)),
                        jax.nn.gelu(hid))
    y_ref[...] = _dot(act, w2) + b2_ref[pl.ds(e, 1), :]


def _ffn_wave(x_rows, w1, b1, w2, b2, off, y_prev=None):
    def body(*refs):
        _ffn_wave_body(off, *refs[:5], refs[-1])

    in_specs = [
        pl.BlockSpec((CAP, C), lambda e: (e, 0)),
        pl.BlockSpec((1, C, HID), lambda e: (e + off, 0, 0)),
        pl.BlockSpec((NE, HID), lambda e: (0, 0)),
        pl.BlockSpec((1, HID, C), lambda e: (e + off, 0, 0)),
        pl.BlockSpec((NE, C), lambda e: (0, 0)),
    ]
    args = [x_rows, w1, b1, w2, b2]
    aliases = {}
    if y_prev is not None:
        in_specs.append(pl.BlockSpec(memory_space=pl.ANY))
        args.append(y_prev)
        aliases = {5: 0}
    return pl.pallas_call(
        body,
        grid=(NEW_,),
        in_specs=in_specs,
        out_specs=pl.BlockSpec((CAP, C), lambda e: (e + off, 0)),
        out_shape=jax.ShapeDtypeStruct((G, C), _F32),
        input_output_aliases=aliases,
    )(*args)


def _combine_body(x1_ref, d_ref, o_ref):
    o_ref[...] = x1_ref[...] + 0.5 * (d_ref[0] + d_ref[1])


def _combine(x1, d):
    blk = pl.BlockSpec((BT, C), lambda i: (i, 0))
    return pl.pallas_call(
        _combine_body,
        grid=(T // BT,),
        in_specs=[blk, pl.BlockSpec((2, BT, C), lambda i: (0, i, 0))],
        out_specs=blk,
        out_shape=jax.ShapeDtypeStruct((T, C), _F32),
    )(x1, d.reshape(2, T, C))


def _sc_gather(data, idx, n_rows):
    width = data.shape[1]
    bpw = n_rows // NW
    mesh = plsc.VectorSubcoreMesh(core_axis_name="c", subcore_axis_name="s")

    @functools.partial(
        pl.kernel, mesh=mesh,
        out_type=jax.ShapeDtypeStruct((n_rows, width), data.dtype),
        scratch_types=[pltpu.VMEM((bpw,), jnp.int32),
                       pltpu.VMEM((bpw, width), data.dtype),
                       pltpu.SemaphoreType.DMA])
    def k(data_hbm, idx_hbm, out_hbm, idx_v, rows_v, sem):
        wid = jax.lax.axis_index("s") * 2 + jax.lax.axis_index("c")
        base = wid * bpw
        pltpu.sync_copy(idx_hbm.at[pl.ds(base, bpw)], idx_v)
        pltpu.async_copy(data_hbm.at[idx_v], rows_v, sem).wait()
        pltpu.sync_copy(rows_v, out_hbm.at[pl.ds(base, bpw)])

    return k(data, idx)


def kernel(x, v_first, capital_shares, params, step, warmup_steps, use_market,
           training):
    p = params
    x2d = x.reshape(T, C)

    x1, v, hh = _prologue(x2d, p)

    hh2 = hh.reshape(2 * T, C)
    xa = _sc_gather(hh2, jnp.asarray(_GIDX_A), HALF)
    xb = _sc_gather(hh2, jnp.asarray(_GIDX_B), HALF)
    ya = _ffn_wave(xa, p['W1'], p['b1'], p['W2'], p['b2'], 0)
    y = _ffn_wave(xb, p['W1'], p['b1'], p['W2'], p['b2'], NEW_, y_prev=ya)

    d = _sc_gather(y, jnp.asarray(_INV), 2 * T)

    x2 = _combine(x1, d)
    return (x2.reshape(B, T, C), v.reshape(B, T, C))
